# baseline (speedup 1.0000x reference)
import functools

import jax
import jax.numpy as jnp
from jax import lax
from jax.experimental import pallas as pl
from jax.experimental.pallas import tpu as pltpu
from jax.experimental.pallas import tpu_sc as plsc

N_NODES = 10000
N_EDGES = 320000
D_FEAT = 128
HIDDEN = 64
TEMP = 5.0
SAMPLE_BIAS = 0.0

NC = 2
NS = 16
NW = NC * NS
PER_W = N_EDGES // NW
B = 80
NCHUNK = PER_W // B
NG = B // 16


def _prep_body(nid_ref, emb_ref, w1_ref, b1_ref, w2_ref, b2_ref, u_ref,
               t1_ref, t2_ref, a_ref, w2b_ref):
    emb = emb_ref[...]
    w1a = w1_ref[0:D_FEAT, :]
    w1b = w1_ref[D_FEAT:2 * D_FEAT, :]
    w1c = w1_ref[2 * D_FEAT:3 * D_FEAT, :]
    nid = nid_ref[0]
    center = emb_ref[pl.ds(nid, 1), :]
    c = jnp.dot(center, w1c, preferred_element_type=jnp.float32) + b1_ref[...]
    t1_ref[...] = jnp.dot(emb, w1a, preferred_element_type=jnp.float32) + c
    t2_ref[...] = jnp.dot(emb, w1b, preferred_element_type=jnp.float32)
    bias = SAMPLE_BIAS + 0.0001
    u = u_ref[...]
    eps = (bias - (1.0 - bias)) * u + (1.0 - bias)
    b2 = b2_ref[0]
    a_ref[...] = jnp.exp((jnp.log(1.0 - eps) - jnp.log(eps) - b2) / TEMP)
    w2b_ref[...] = jnp.broadcast_to(w2_ref[...], (HIDDEN, 16))


_prep = pl.pallas_call(
    _prep_body,
    out_shape=[
        jax.ShapeDtypeStruct((N_NODES, HIDDEN), jnp.float32),
        jax.ShapeDtypeStruct((N_NODES, HIDDEN), jnp.float32),
        jax.ShapeDtypeStruct((N_EDGES // D_FEAT, D_FEAT), jnp.float32),
        jax.ShapeDtypeStruct((HIDDEN, 16), jnp.float32),
    ],
    in_specs=[
        pl.BlockSpec(memory_space=pltpu.SMEM),
        pl.BlockSpec(memory_space=pltpu.VMEM),
        pl.BlockSpec(memory_space=pltpu.VMEM),
        pl.BlockSpec(memory_space=pltpu.VMEM),
        pl.BlockSpec(memory_space=pltpu.VMEM),
        pl.BlockSpec(memory_space=pltpu.SMEM),
        pl.BlockSpec(memory_space=pltpu.VMEM),
    ],
)


NBUF = 5
NOUTER = NCHUNK // NBUF
KUNROLL = 4


STAGE = N_NODES // NS


def _sc_body(t1_hbm, t2_hbm, src_hbm, dst_hbm, a_hbm, w2b_hbm, out_hbm,
             src_v, dst_v, a_v, out_v, w2b_v, r_v, *sems):
    sem1 = sems[:NBUF]
    sem2 = sems[NBUF:]
    sid = lax.axis_index("s")
    wid = sid * NC + lax.axis_index("c")
    base = wid * PER_W
    pltpu.sync_copy(src_hbm.at[pl.ds(base, PER_W)], src_v)
    pltpu.sync_copy(dst_hbm.at[pl.ds(base, PER_W)], dst_v)
    pltpu.sync_copy(a_hbm.at[pl.ds(base, PER_W)], a_v)
    pltpu.sync_copy(w2b_hbm, w2b_v)

    iota16 = lax.iota(jnp.int32, 16)

    def issue_g1(j, b):
        pltpu.async_copy(
            t1_hbm.at[src_v.at[pl.ds(j * B, B)]], r_v.at[b], sem1[b])

    def wait_g1(j, b):
        pltpu.make_async_copy(
            t1_hbm.at[src_v.at[pl.ds(j * B, B)]], r_v.at[b], sem1[b]).wait()

    def issue_g2(j, b):
        pltpu.async_copy(
            t2_hbm.at[dst_v.at[pl.ds(j * B, B)]], r_v.at[b], sem2[b],
            add=True)

    def wait_g2(j, b):
        pltpu.make_async_copy(
            t2_hbm.at[dst_v.at[pl.ds(j * B, B)]], r_v.at[b], sem2[b]).wait()

    for j in range(NBUF):
        issue_g1(j, j)
    for j in range(3):
        wait_g1(j, j)
        issue_g2(j, j)

    def outer(o, _):
        for b in range(NBUF):
            i = o * NBUF + b
            j3 = i + 3
            b3 = (b + 3) % NBUF

            @pl.when(j3 < NCHUNK)
            def _():
                wait_g1(j3, b3)
                issue_g2(j3, b3)

            wait_g2(i, b)
            off = i * B

            def kblock(kk, accs):
                for dk in range(KUNROLL):
                    k = kk * KUNROLL + dk
                    w2k = w2b_v[k]
                    accs = tuple(
                        accs[g] + jnp.maximum(
                            plsc.load_gather(
                                r_v.at[b],
                                [iota16 + g * 16,
                                 jnp.full((16,), k, jnp.int32)]),
                            0.0) * w2k
                        for g in range(NG))
                return accs

            accs = lax.fori_loop(
                0, HIDDEN // KUNROLL, kblock,
                tuple(jnp.zeros((16,), jnp.float32) for _ in range(NG)),
                unroll=False)
            for g in range(NG):
                av = a_v[pl.ds(off + g * 16, 16)]
                out_v[pl.ds(off + g * 16, 16)] = (
                    1.0 / (1.0 + av * jnp.exp(accs[g] * (-1.0 / TEMP))))

            @pl.when(i + NBUF < NCHUNK)
            def _():
                issue_g1(i + NBUF, b)
        return ()

    lax.fori_loop(0, NOUTER, outer, (), unroll=False)
    pltpu.sync_copy(out_v, out_hbm.at[pl.ds(base, PER_W)])


_sc = functools.partial(
    pl.kernel,
    out_type=jax.ShapeDtypeStruct((N_EDGES,), jnp.float32),
    mesh=plsc.VectorSubcoreMesh(
        core_axis_name="c", subcore_axis_name="s",
        num_cores=NC, num_subcores=NS),
    compiler_params=pltpu.CompilerParams(
        needs_layout_passes=False, use_tc_tiling_on_sc=False),
    scratch_types=[
        pltpu.VMEM((PER_W,), jnp.int32),
        pltpu.VMEM((PER_W,), jnp.int32),
        pltpu.VMEM((PER_W,), jnp.float32),
        pltpu.VMEM((PER_W,), jnp.float32),
        pltpu.VMEM((HIDDEN, 16), jnp.float32),
        pltpu.VMEM((NBUF, B, HIDDEN), jnp.float32),
    ] + [pltpu.SemaphoreType.DMA] * (2 * NBUF),
)(_sc_body)


def kernel(embeds, edge_index, u, W1, b1, W2, b2, node_idx):
    src = edge_index[0]
    dst = edge_index[1]
    nid = jnp.asarray(node_idx, jnp.int32).reshape(1)
    u2 = u.reshape(N_EDGES // D_FEAT, D_FEAT)
    t1, t2, a2, w2b = _prep(nid, embeds, W1, b1, W2, b2, u2)
    a = a2.reshape(N_EDGES)
    return _sc(t1, t2, src, dst, a, w2b)


# speedup vs baseline: 2.3888x; 2.3888x over previous
import functools

import jax
import jax.numpy as jnp
from jax import lax
from jax.experimental import pallas as pl
from jax.experimental.pallas import tpu as pltpu
from jax.experimental.pallas import tpu_sc as plsc

N_NODES = 10000
N_EDGES = 320000
D_FEAT = 128
HIDDEN = 64
TEMP = 5.0
SAMPLE_BIAS = 0.0

NC = 2
NS = 16
NW = NC * NS
PER_W = N_EDGES // NW
B = 80
NCHUNK = PER_W // B
NG = B // 16


def _prep_body(nid_ref, emb_ref, w1_ref, b1_ref, w2_ref, b2_ref, u_ref,
               t1_ref, t2_ref, a_ref):
    emb = emb_ref[...]
    w1a = w1_ref[0:D_FEAT, :]
    w1b = w1_ref[D_FEAT:2 * D_FEAT, :]
    w1c = w1_ref[2 * D_FEAT:3 * D_FEAT, :]
    nid = nid_ref[0]
    center = emb_ref[pl.ds(nid, 1), :]
    c = jnp.dot(center, w1c, preferred_element_type=jnp.float32) + b1_ref[...]
    t1_ref[...] = (
        jnp.dot(emb, w1a, preferred_element_type=jnp.float32) + c
    ).astype(jnp.bfloat16)
    t2_ref[...] = jnp.dot(
        emb, w1b, preferred_element_type=jnp.float32).astype(jnp.bfloat16)
    bias = SAMPLE_BIAS + 0.0001
    u = u_ref[...]
    eps = (bias - (1.0 - bias)) * u + (1.0 - bias)
    b2 = b2_ref[0]
    a_ref[...] = jnp.exp((jnp.log(1.0 - eps) - jnp.log(eps) - b2) / TEMP)


_prep = pl.pallas_call(
    _prep_body,
    out_shape=[
        jax.ShapeDtypeStruct((N_NODES, HIDDEN), jnp.bfloat16),
        jax.ShapeDtypeStruct((N_NODES, HIDDEN), jnp.bfloat16),
        jax.ShapeDtypeStruct((N_EDGES // D_FEAT, D_FEAT), jnp.float32),
    ],
    in_specs=[
        pl.BlockSpec(memory_space=pltpu.SMEM),
        pl.BlockSpec(memory_space=pltpu.VMEM),
        pl.BlockSpec(memory_space=pltpu.VMEM),
        pl.BlockSpec(memory_space=pltpu.VMEM),
        pl.BlockSpec(memory_space=pltpu.VMEM),
        pl.BlockSpec(memory_space=pltpu.SMEM),
        pl.BlockSpec(memory_space=pltpu.VMEM),
    ],
)


NBUF = 5
NOUTER = NCHUNK // NBUF
KUNROLL = 4


STAGE = N_NODES // NS


def _sc_body(t1_hbm, t2_hbm, ei_hbm, a_hbm, w2f_hbm, out_hbm,
             src_v, dst_v, a_v, out_v, w2f_v, r_v, *sems):
    sem1 = sems[:NBUF]
    sem2 = sems[NBUF:]
    sid = lax.axis_index("s")
    wid = sid * NC + lax.axis_index("c")
    base = wid * PER_W
    pltpu.sync_copy(ei_hbm.at[0, pl.ds(base, PER_W)], src_v)
    pltpu.sync_copy(ei_hbm.at[1, pl.ds(base, PER_W)], dst_v)
    pltpu.sync_copy(a_hbm.at[pl.ds(base, PER_W)], a_v)
    pltpu.sync_copy(w2f_hbm, w2f_v)

    iota16 = lax.iota(jnp.int32, 16)
    rotidx = {d: (iota16 + d) & 15 for d in (8, 4, 2, 1)}
    wregs = [w2f_v[pl.ds(16 * t, 16)] for t in range(4)]

    def issue_g1(j, b):
        pltpu.async_copy(
            t1_hbm.at[src_v.at[pl.ds(j * B, B)]], r_v.at[b], sem1[b])

    def wait_g1(j, b):
        pltpu.make_async_copy(
            t1_hbm.at[src_v.at[pl.ds(j * B, B)]], r_v.at[b], sem1[b]).wait()

    def issue_g2(j, b):
        pltpu.async_copy(
            t2_hbm.at[dst_v.at[pl.ds(j * B, B)]], r_v.at[b], sem2[b],
            add=True)

    def wait_g2(j, b):
        pltpu.make_async_copy(
            t2_hbm.at[dst_v.at[pl.ds(j * B, B)]], r_v.at[b], sem2[b]).wait()

    for j in range(NBUF):
        issue_g1(j, j)
    for j in range(3):
        wait_g1(j, j)
        issue_g2(j, j)

    def outer(o, _):
        for b in range(NBUF):
            i = o * NBUF + b
            j3 = i + 3
            b3 = (b + 3) % NBUF

            @pl.when(j3 < NCHUNK)
            def _():
                wait_g1(j3, b3)
                issue_g2(j3, b3)

            wait_g2(i, b)
            off = i * B

            def group(g, _):
                s_vec = jnp.zeros((16,), jnp.float32)
                for e in range(16):
                    row = g * 16 + e
                    x = None
                    for t in range(2):
                        z32 = r_v.at[b][row, pl.ds(32 * t, 32)]
                        za, zb = plsc.unpack(
                            z32, format=plsc.PackFormat.INTERLEAVED)
                        p = (jnp.maximum(za, 0.0) * wregs[2 * t]
                             + jnp.maximum(zb, 0.0) * wregs[2 * t + 1])
                        x = p if x is None else x + p
                    for d in (8, 4, 2, 1):
                        x = x + jnp.take(x, rotidx[d])
                    s_vec = jnp.where(iota16 == e, x, s_vec)
                av = a_v[pl.ds(off + g * 16, 16)]
                out_v[pl.ds(off + g * 16, 16)] = (
                    1.0 / (1.0 + av * jnp.exp(s_vec * (-1.0 / TEMP))))
                return ()

            lax.fori_loop(0, NG, group, (), unroll=False)

            @pl.when(i + NBUF < NCHUNK)
            def _():
                issue_g1(i + NBUF, b)
        return ()

    lax.fori_loop(0, NOUTER, outer, (), unroll=False)
    pltpu.sync_copy(out_v, out_hbm.at[pl.ds(base, PER_W)])


_sc = functools.partial(
    pl.kernel,
    out_type=jax.ShapeDtypeStruct((N_EDGES,), jnp.float32),
    mesh=plsc.VectorSubcoreMesh(
        core_axis_name="c", subcore_axis_name="s",
        num_cores=NC, num_subcores=NS),
    compiler_params=pltpu.CompilerParams(
        needs_layout_passes=False, use_tc_tiling_on_sc=False),
    scratch_types=[
        pltpu.VMEM((PER_W,), jnp.int32),
        pltpu.VMEM((PER_W,), jnp.int32),
        pltpu.VMEM((PER_W,), jnp.float32),
        pltpu.VMEM((PER_W,), jnp.float32),
        pltpu.VMEM((HIDDEN,), jnp.float32),
        pltpu.VMEM((NBUF, B, HIDDEN), jnp.bfloat16),
    ] + [pltpu.SemaphoreType.DMA] * (2 * NBUF),
)(_sc_body)


def kernel(embeds, edge_index, u, W1, b1, W2, b2, node_idx):
    nid = jnp.asarray(node_idx, jnp.int32).reshape(1)
    u2 = u.reshape(N_EDGES // D_FEAT, D_FEAT)
    t1, t2, a2 = _prep(nid, embeds, W1, b1, W2, b2, u2)
    a = a2.reshape(N_EDGES)
    w2f = W2.reshape(HIDDEN)
    w2r = jnp.concatenate(
        [w2f[0:32:2], w2f[1:32:2], w2f[32:64:2], w2f[33:64:2]])
    return _sc(t1, t2, edge_index, a, w2r)
